# baseline (speedup 1.0000x reference)
import functools

import jax
import jax.numpy as jnp
from jax import lax
from jax.experimental import pallas as pl
from jax.experimental.pallas import tpu as pltpu
from jax.experimental.pallas import tpu_sc as plsc

_D = 64
_BT = 128
_PIPE = 4


def _make_gather(n_h: int, n_b: int):
    info = plsc.get_sparse_core_info()
    nc, ns = info.num_cores, info.num_subcores
    nw = nc * ns
    assert n_b == nw * _BT and n_h % _PIPE == 0
    steps = n_h // _PIPE

    mesh = plsc.VectorSubcoreMesh(core_axis_name="c", subcore_axis_name="s")

    @functools.partial(
        pl.kernel,
        mesh=mesh,
        out_type=jax.ShapeDtypeStruct((n_h, _D // 8, nw, 8, _BT), jnp.float32),
        scratch_types=[
            pltpu.VMEM((n_h, _BT), jnp.int32)] + [
            pltpu.VMEM((_BT,), jnp.int32) for _ in range(_PIPE)] + [
            pltpu.VMEM((_BT, _D), jnp.float32) for _ in range(_PIPE)] + [
            pltpu.VMEM((_D // 8, 8, _BT), jnp.float32) for _ in range(_PIPE)] + [
            pltpu.SemaphoreType.DMA for _ in range(2 * _PIPE)],
        compiler_params=pltpu.CompilerParams(
            use_tc_tiling_on_sc=False, needs_layout_passes=False),
    )
    def gather_kernel(xt_hbm, table_hbm, out_hbm, xcol, *bufs):
        idxs = bufs[:_PIPE]
        rows = bufs[_PIPE:2 * _PIPE]
        tiles = bufs[2 * _PIPE:3 * _PIPE]
        sg = bufs[3 * _PIPE:4 * _PIPE]
        sw = bufs[4 * _PIPE:5 * _PIPE]
        bt = lax.axis_index("s") * nc + lax.axis_index("c")

        pltpu.sync_copy(xt_hbm.at[:, pl.ds(bt * _BT, _BT)], xcol)

        def fire(h, k):
            for i in range(_BT // 16):
                s = pl.ds(i * 16, 16)
                idxs[k][s] = ~xcol[h, s]
            pltpu.async_copy(table_hbm.at[idxs[k]], rows[k], sg[k])

        def wait_gather(k):
            pltpu.make_async_copy(table_hbm.at[idxs[k]], rows[k], sg[k]).wait()

        def transpose(k):
            iota16 = lax.iota(jnp.int32, 16)
            rowvs = [iota16 + b0 for b0 in range(0, _BT, 16)]

            def dcol_body(i, carry):
                for u in range(2):
                    dcol = i * 2 + u
                    dt = dcol // 8
                    dsub = dcol % 8
                    colv = jnp.zeros((16,), jnp.int32) + dcol
                    for j in range(_BT // 16):
                        v = plsc.load_gather(rows[k], [rowvs[j], colv])
                        tiles[k][dt, dsub, pl.ds(j * 16, 16)] = v
                return carry

            lax.fori_loop(0, _D // 2, dcol_body, 0)

        def fire_wb(h, k):
            pltpu.async_copy(tiles[k], out_hbm.at[h, :, bt], sw[k])

        def wait_wb(k):
            pltpu.make_async_copy(tiles[k], out_hbm.at[0, :, bt], sw[k]).wait()

        for k in range(_PIPE):
            fire(k, k)

        def step_body(t, carry):
            for k in range(_PIPE):
                h = _PIPE * t + k
                wait_gather(k)

                @pl.when(t > 0)
                def _():
                    wait_wb(k)

                transpose(k)
                fire_wb(h, k)

                @pl.when(t + 1 < steps)
                def _():
                    fire(h + _PIPE, k)

            return carry

        lax.fori_loop(0, steps, step_body, 0)
        for k in range(_PIPE):
            wait_wb(k)

    return gather_kernel


def kernel(x, table):
    b, h = x.shape
    xt = jnp.transpose(x)
    o5 = _make_gather(h, b)(xt, table)
    return o5.transpose(2, 4, 0, 1, 3).reshape(b, h, _D)


# speedup vs baseline: 1.3229x; 1.3229x over previous
import functools

import jax
import jax.numpy as jnp
from jax import lax
from jax.experimental import pallas as pl
from jax.experimental.pallas import tpu as pltpu
from jax.experimental.pallas import tpu_sc as plsc

_D = 64
_BT = 128
_PIPE = 4


def _make_gather(n_h: int, n_b: int):
    info = plsc.get_sparse_core_info()
    nc, ns = info.num_cores, info.num_subcores
    nw = nc * ns
    assert n_b == nw * _BT and n_h % _PIPE == 0
    steps = n_h // _PIPE

    mesh = plsc.VectorSubcoreMesh(core_axis_name="c", subcore_axis_name="s")

    @functools.partial(
        pl.kernel,
        mesh=mesh,
        out_type=jax.ShapeDtypeStruct((n_h, _D // 8, nw, 8, _BT), jnp.float32),
        scratch_types=[
            pltpu.VMEM((n_h, _BT), jnp.int32)] + [
            pltpu.VMEM((_BT,), jnp.int32) for _ in range(_PIPE)] + [
            pltpu.VMEM((_BT, _D), jnp.float32) for _ in range(_PIPE)] + [
            pltpu.VMEM((_D // 8, 8, _BT), jnp.float32) for _ in range(_PIPE)] + [
            pltpu.SemaphoreType.DMA for _ in range(2 * _PIPE)],
        compiler_params=pltpu.CompilerParams(
            use_tc_tiling_on_sc=False, needs_layout_passes=False),
    )
    def gather_kernel(xt_hbm, table_hbm, out_hbm, xcol, *bufs):
        idxs = bufs[:_PIPE]
        rows = bufs[_PIPE:2 * _PIPE]
        tiles = bufs[2 * _PIPE:3 * _PIPE]
        sg = bufs[3 * _PIPE:4 * _PIPE]
        sw = bufs[4 * _PIPE:5 * _PIPE]
        bt = lax.axis_index("s") * nc + lax.axis_index("c")

        pltpu.sync_copy(xt_hbm.at[:, pl.ds(bt * _BT, _BT)], xcol)

        def fire(h, k):
            for i in range(_BT // 16):
                s = pl.ds(i * 16, 16)
                idxs[k][s] = ~xcol[h, s]
            pltpu.async_copy(table_hbm.at[idxs[k]], rows[k], sg[k])

        def wait_gather(k):
            pltpu.make_async_copy(table_hbm.at[idxs[k]], rows[k], sg[k]).wait()

        def transpose(k):
            iota16 = lax.iota(jnp.int32, 16)
            rowvs = [iota16 + b0 for b0 in range(0, _BT, 16)]

            @plsc.parallel_loop(0, _D, 1, unroll=4)
            def dcol_body(dcol):
                dt = dcol // 8
                dsub = dcol % 8
                colv = jnp.zeros((16,), jnp.int32) + dcol
                vs = [plsc.load_gather(rows[k], [rowvs[j], colv])
                      for j in range(_BT // 16)]
                for j in range(_BT // 16):
                    tiles[k][dt, dsub, pl.ds(j * 16, 16)] = vs[j]

        def fire_wb(h, k):
            pltpu.async_copy(tiles[k], out_hbm.at[h, :, bt], sw[k])

        def wait_wb(k):
            pltpu.make_async_copy(tiles[k], out_hbm.at[0, :, bt], sw[k]).wait()

        for k in range(_PIPE):
            fire(k, k)

        def step_body(t, carry):
            for k in range(_PIPE):
                h = _PIPE * t + k
                wait_gather(k)

                @pl.when(t > 0)
                def _():
                    wait_wb(k)

                transpose(k)
                fire_wb(h, k)

                @pl.when(t + 1 < steps)
                def _():
                    fire(h + _PIPE, k)

            return carry

        lax.fori_loop(0, steps, step_body, 0)
        for k in range(_PIPE):
            wait_wb(k)

    return gather_kernel


def kernel(x, table):
    b, h = x.shape
    xt = jnp.transpose(x)
    o5 = _make_gather(h, b)(xt, table)
    return o5.transpose(2, 4, 0, 1, 3).reshape(b, h, _D)


# speedup vs baseline: 4.5222x; 3.4185x over previous
import functools

import jax
import jax.numpy as jnp
from jax import lax
from jax.experimental import pallas as pl
from jax.experimental.pallas import tpu as pltpu
from jax.experimental.pallas import tpu_sc as plsc

_D = 64
_BT = 128
_PIPE = 4


def _make_gather(n_h: int, n_b: int):
    info = plsc.get_sparse_core_info()
    nc, ns = info.num_cores, info.num_subcores
    nw = nc * ns
    assert n_b == nw * _BT and n_h % _PIPE == 0
    steps = n_h // _PIPE

    mesh = plsc.VectorSubcoreMesh(core_axis_name="c", subcore_axis_name="s")

    @functools.partial(
        pl.kernel,
        mesh=mesh,
        out_type=jax.ShapeDtypeStruct((n_h, _D // 8, nw, 8, _BT), jnp.float32),
        scratch_types=[
            pltpu.VMEM((n_h, _BT), jnp.int32)] + [
            pltpu.VMEM((_BT,), jnp.int32) for _ in range(_PIPE)] + [
            pltpu.VMEM((_BT, _D), jnp.float32) for _ in range(_PIPE)] + [
            pltpu.VMEM((_D // 8, 8, _BT + 1), jnp.float32) for _ in range(_PIPE)] + [
            pltpu.SemaphoreType.DMA for _ in range(2 * _PIPE)],
        compiler_params=pltpu.CompilerParams(
            use_tc_tiling_on_sc=False, needs_layout_passes=False),
    )
    def gather_kernel(xt_hbm, table_hbm, out_hbm, xcol, *bufs):
        idxs = bufs[:_PIPE]
        rows = bufs[_PIPE:2 * _PIPE]
        tiles = bufs[2 * _PIPE:3 * _PIPE]
        sg = bufs[3 * _PIPE:4 * _PIPE]
        sw = bufs[4 * _PIPE:5 * _PIPE]
        bt = lax.axis_index("s") * nc + lax.axis_index("c")

        pltpu.sync_copy(xt_hbm.at[:, pl.ds(bt * _BT, _BT)], xcol)

        def fire(h, k):
            for i in range(_BT // 16):
                s = pl.ds(i * 16, 16)
                idxs[k][s] = ~xcol[h, s]
            pltpu.async_copy(table_hbm.at[idxs[k]], rows[k], sg[k])

        def wait_gather(k):
            pltpu.make_async_copy(table_hbm.at[idxs[k]], rows[k], sg[k]).wait()

        iota16 = lax.iota(jnp.int32, 16)
        dtvs = [(iota16 + d0) >> 3 for d0 in range(0, _D, 16)]
        dsvs = [(iota16 + d0) & 7 for d0 in range(0, _D, 16)]

        def transpose(k):
            @plsc.parallel_loop(0, _BT, 1, unroll=2)
            def bl_body(bl):
                blv = jnp.zeros((16,), jnp.int32) + bl
                vs = [rows[k][bl, pl.ds(d0, 16)] for d0 in range(0, _D, 16)]
                for j in range(_D // 16):
                    plsc.store_scatter(tiles[k], [dtvs[j], dsvs[j], blv],
                                       vs[j])

        def fire_wb(h, k):
            pltpu.async_copy(tiles[k].at[:, :, pl.ds(0, _BT)],
                             out_hbm.at[h, :, bt], sw[k])

        def wait_wb(k):
            pltpu.make_async_copy(tiles[k].at[:, :, pl.ds(0, _BT)],
                                  out_hbm.at[0, :, bt], sw[k]).wait()

        for k in range(_PIPE):
            fire(k, k)

        def step_body(t, carry):
            for k in range(_PIPE):
                h = _PIPE * t + k
                wait_gather(k)

                @pl.when(t > 0)
                def _():
                    wait_wb(k)

                transpose(k)
                fire_wb(h, k)

                @pl.when(t + 1 < steps)
                def _():
                    fire(h + _PIPE, k)

            return carry

        lax.fori_loop(0, steps, step_body, 0)
        for k in range(_PIPE):
            wait_wb(k)

    return gather_kernel


def kernel(x, table):
    b, h = x.shape
    xt = jnp.transpose(x)
    o5 = _make_gather(h, b)(xt, table)
    return o5.transpose(2, 4, 0, 1, 3).reshape(b, h, _D)


# speedup vs baseline: 4.5298x; 1.0017x over previous
import functools

import jax
import jax.numpy as jnp
from jax import lax
from jax.experimental import pallas as pl
from jax.experimental.pallas import tpu as pltpu
from jax.experimental.pallas import tpu_sc as plsc

_D = 64
_BT = 128
_PIPE = 4


def _make_gather(n_h: int, n_b: int):
    info = plsc.get_sparse_core_info()
    nc, ns = info.num_cores, info.num_subcores
    nw = nc * ns
    assert n_b == nw * _BT and n_h % _PIPE == 0
    steps = n_h // _PIPE

    mesh = plsc.VectorSubcoreMesh(core_axis_name="c", subcore_axis_name="s")

    @functools.partial(
        pl.kernel,
        mesh=mesh,
        out_type=jax.ShapeDtypeStruct((n_h, _D // 8, nw, 8, _BT), jnp.float32),
        scratch_types=[
            pltpu.VMEM((n_h, _BT), jnp.int32)] + [
            pltpu.VMEM((_BT,), jnp.int32) for _ in range(_PIPE)] + [
            pltpu.VMEM((_BT, _D), jnp.float32) for _ in range(_PIPE)] + [
            pltpu.VMEM((_D // 8, 8, _BT + 1), jnp.float32) for _ in range(_PIPE)] + [
            pltpu.SemaphoreType.DMA for _ in range(2 * _PIPE)],
        compiler_params=pltpu.CompilerParams(
            use_tc_tiling_on_sc=False, needs_layout_passes=False),
    )
    def gather_kernel(xt_hbm, table_hbm, out_hbm, xcol, *bufs):
        idxs = bufs[:_PIPE]
        rows = bufs[_PIPE:2 * _PIPE]
        tiles = bufs[2 * _PIPE:3 * _PIPE]
        sg = bufs[3 * _PIPE:4 * _PIPE]
        sw = bufs[4 * _PIPE:5 * _PIPE]
        bt = lax.axis_index("s") * nc + lax.axis_index("c")

        pltpu.sync_copy(xt_hbm.at[:, pl.ds(bt * _BT, _BT)], xcol)

        def fire(h, k):
            for i in range(_BT // 16):
                s = pl.ds(i * 16, 16)
                idxs[k][s] = ~xcol[h, s]
            pltpu.async_copy(table_hbm.at[idxs[k]], rows[k], sg[k])

        def wait_gather(k):
            pltpu.make_async_copy(table_hbm.at[idxs[k]], rows[k], sg[k]).wait()

        iota16 = lax.iota(jnp.int32, 16)
        dtvs = [(iota16 + d0) >> 3 for d0 in range(0, _D, 16)]
        dsvs = [(iota16 + d0) & 7 for d0 in range(0, _D, 16)]

        def transpose(k):
            @plsc.parallel_loop(0, _BT, 1, unroll=4)
            def bl_body(bl):
                blv = jnp.zeros((16,), jnp.int32) + bl
                vs = [rows[k][bl, pl.ds(d0, 16)] for d0 in range(0, _D, 16)]
                for j in range(_D // 16):
                    plsc.store_scatter(tiles[k], [dtvs[j], dsvs[j], blv],
                                       vs[j])

        def fire_wb(h, k):
            pltpu.async_copy(tiles[k].at[:, :, pl.ds(0, _BT)],
                             out_hbm.at[h, :, bt], sw[k])

        def wait_wb(k):
            pltpu.make_async_copy(tiles[k].at[:, :, pl.ds(0, _BT)],
                                  out_hbm.at[0, :, bt], sw[k]).wait()

        for k in range(_PIPE):
            fire(k, k)

        def step_body(t, carry):
            for k in range(_PIPE):
                h = _PIPE * t + k
                wait_gather(k)

                @pl.when(t > 0)
                def _():
                    wait_wb(k)

                transpose(k)
                fire_wb(h, k)

                @pl.when(t + 1 < steps)
                def _():
                    fire(h + _PIPE, k)

            return carry

        lax.fori_loop(0, steps, step_body, 0)
        for k in range(_PIPE):
            wait_wb(k)

    return gather_kernel


def kernel(x, table):
    b, h = x.shape
    xt = jnp.transpose(x)
    o5 = _make_gather(h, b)(xt, table)
    return o5.transpose(2, 4, 0, 1, 3).reshape(b, h, _D)
